# Initial kernel scaffold; baseline (speedup 1.0000x reference)
import jax
import jax.numpy as jnp
from jax.experimental import pallas as pl


def kernel(edge_index, Wp1, bp1, Ws1, Wn1, b1, Wp2, bp2, Ws2, Wn2, b2):
    raise NotImplementedError("write your pallas kernel here")


# speedup vs baseline: 1.2886x; 1.2886x over previous
import functools

import jax
import jax.numpy as jnp
from jax import lax
from jax.experimental import pallas as pl
from jax.experimental.pallas import tpu as pltpu
from jax.experimental.pallas import tpu_sc as plsc

N = 10000
E = 320000
H = 128

_NC = 2
_NS = 16
_NTILES = _NC * _NS
_RPT = H // _NTILES
_CHUNK = 8000
_NCHUNK = E // _CHUNK
_GROUPS = _CHUNK // 16

_BLK = 1000
_NBLK = N // _BLK


def _dotT(w, x):
    return lax.dot_general(w, x, (((0,), (0,)), ((), ())),
                           preferred_element_type=jnp.float32)


def _sc_segmax_body(p_hbm, src_hbm, dst_hbm, out_hbm, pool_v, acc_v,
                    src_v, dst_v):
    wid = lax.axis_index("s") * _NC + lax.axis_index("c")
    base = wid * _RPT * N

    pltpu.sync_copy(p_hbm.at[pl.ds(base, _RPT * N)], pool_v)

    zero16 = jnp.zeros((16,), jnp.float32)

    def _zero(i, carry):
        acc_v[pl.ds(i * 16, 16)] = zero16
        return carry

    lax.fori_loop(0, _RPT * N // 16, _zero, 0)

    offs = [jnp.full((16,), r * N, jnp.int32) for r in range(_RPT)]

    def _chunk(ci, carry):
        pltpu.sync_copy(src_hbm.at[pl.ds(ci * _CHUNK, _CHUNK)], src_v)
        pltpu.sync_copy(dst_hbm.at[pl.ds(ci * _CHUNK, _CHUNK)], dst_v)

        def _group(gi, c2):
            sid = src_v[pl.ds(gi * 16, 16)]
            did = dst_v[pl.ds(gi * 16, 16)]
            sidx = [sid + offs[r] for r in range(_RPT)]
            didx = [did + offs[r] for r in range(_RPT)]
            news = []
            flag = jnp.bool_(False)
            for r in range(_RPT):
                v = plsc.load_gather(pool_v, [sidx[r]])
                cur = plsc.load_gather(acc_v, [didx[r]])
                new = jnp.maximum(cur, v)
                plsc.store_scatter(acc_v, [didx[r]], new)
                news.append(new)
            for r in range(_RPT):
                chk = plsc.load_gather(acc_v, [didx[r]])
                flag = flag | jnp.any(chk < news[r])

            def _retry_cond(f):
                return f

            def _retry(_f):
                f2 = jnp.bool_(False)
                for r in range(_RPT):
                    chk = plsc.load_gather(acc_v, [didx[r]])
                    need = chk < news[r]
                    plsc.store_scatter(acc_v, [didx[r]], news[r],
                                       mask=need)
                    f2 = f2 | jnp.any(need)
                return f2

            lax.while_loop(_retry_cond, _retry, flag)
            return c2

        lax.fori_loop(0, _GROUPS, _group, 0)
        return carry

    lax.fori_loop(0, _NCHUNK, _chunk, 0)

    pltpu.sync_copy(acc_v, out_hbm.at[pl.ds(base, _RPT * N)])


@functools.partial(
    pl.kernel,
    out_type=jax.ShapeDtypeStruct((H * N,), jnp.float32),
    mesh=plsc.VectorSubcoreMesh(core_axis_name="c", subcore_axis_name="s"),
    compiler_params=pltpu.CompilerParams(needs_layout_passes=False),
    scratch_types=[
        pltpu.VMEM((_RPT * N,), jnp.float32),
        pltpu.VMEM((_RPT * N,), jnp.float32),
        pltpu.VMEM((_CHUNK,), jnp.int32),
        pltpu.VMEM((_CHUNK,), jnp.int32),
    ],
)
def _sc_segmax_flat(p_hbm, src_hbm, dst_hbm, out_hbm, pool_v, acc_v, src_v,
                    dst_v):
    _sc_segmax_body(p_hbm, src_hbm, dst_hbm, out_hbm, pool_v, acc_v,
                    src_v, dst_v)


def _sc_segmax(p, src, dst):
    return _sc_segmax_flat(p.reshape(H * N), src, dst).reshape(H, N)


def _tc_first_body(s_ref, a_ref, wn_ref, b_ref, wp_ref, bp_ref,
                   g_ref, p_ref):
    g = jax.nn.relu(s_ref[...] + _dotT(wn_ref[...], a_ref[...])
                    + b_ref[...])
    g_ref[...] = g
    p_ref[...] = jax.nn.relu(_dotT(wp_ref[...], g) + bp_ref[...])


def _tc_mid_body(g_ref, a_ref, ws_ref, wn_ref, b_ref, wp_ref, bp_ref,
                 go_ref, p_ref):
    g = jax.nn.relu(_dotT(ws_ref[...], g_ref[...])
                    + _dotT(wn_ref[...], a_ref[...]) + b_ref[...])
    go_ref[...] = g
    p_ref[...] = jax.nn.relu(_dotT(wp_ref[...], g) + bp_ref[...])


def _tc_final_body(g_ref, a_ref, ws_ref, wn_ref, b_ref, out_ref):
    o = (_dotT(ws_ref[...], g_ref[...])
         + _dotT(wn_ref[...], a_ref[...]) + b_ref[...])
    out_ref[...] = o.T


def _tc_first(s, a, wn, b, wp, bp):
    return pl.pallas_call(
        _tc_first_body,
        out_shape=[jax.ShapeDtypeStruct((H, N), jnp.float32)] * 2,
    )(s, a, wn, b, wp, bp)


def _tc_mid(g, a, ws, wn, b, wp, bp):
    return pl.pallas_call(
        _tc_mid_body,
        out_shape=[jax.ShapeDtypeStruct((H, N), jnp.float32)] * 2,
    )(g, a, ws, wn, b, wp, bp)


def _tc_final(g, a, ws, wn, b):
    return pl.pallas_call(
        _tc_final_body,
        out_shape=jax.ShapeDtypeStruct((N, H), jnp.float32),
    )(g, a, ws, wn, b)


def kernel(edge_index, Wp1, bp1, Ws1, Wn1, b1, Wp2, bp2, Ws2, Wn2, b2):
    src = edge_index[0]
    dst = edge_index[1]
    pad = jnp.zeros((H, N - H), jnp.float32)

    p1 = jax.nn.relu(jnp.concatenate([Wp1.T, pad], axis=1) + bp1[:, None])
    s1 = jnp.concatenate([Ws1.T, pad], axis=1)

    a1 = _sc_segmax(p1, src, dst)
    g1, p2 = _tc_first(s1, a1, Wn1, b1[:, None], Wp1, bp1[:, None])
    a2 = _sc_segmax(p2, src, dst)
    g2, p3 = _tc_mid(g1, a2, Ws1, Wn1, b1[:, None], Wp2, bp2[:, None])
    a3 = _sc_segmax(p3, src, dst)
    return _tc_final(g2, a3, Ws2, Wn2, b2[:, None])


# speedup vs baseline: 2.1469x; 1.6661x over previous
import functools

import jax
import jax.numpy as jnp
from jax import lax
from jax.experimental import pallas as pl
from jax.experimental.pallas import tpu as pltpu
from jax.experimental.pallas import tpu_sc as plsc

N = 10000
E = 320000
H = 128

_NC = 2
_NS = 16
_NTILES = _NC * _NS
_RPT = H // _NTILES
_CHUNK = 8000
_NCHUNK = E // _CHUNK
_GROUPS = _CHUNK // 16

_BLK = 1000
_NBLK = N // _BLK


def _dotT(w, x):
    return lax.dot_general(w, x, (((0,), (0,)), ((), ())),
                           preferred_element_type=jnp.float32)


def _sc_segmax_body(p_hbm, src_hbm, dst_hbm, out_hbm, pool_v, acc_v,
                    src_v, dst_v, dup_v):
    wid = lax.axis_index("s") * _NC + lax.axis_index("c")
    base = wid * _RPT * N

    pltpu.sync_copy(p_hbm.at[pl.ds(base, _RPT * N)], pool_v)

    zero16 = jnp.zeros((16,), jnp.float32)

    def _zero(i, carry):
        acc_v[pl.ds(i * 16, 16)] = zero16
        return carry

    lax.fori_loop(0, _RPT * N // 16, _zero, 0)

    offs = [jnp.full((16,), r * N, jnp.int32) for r in range(_RPT)]

    def _chunk(ci, carry):
        pltpu.sync_copy(src_hbm.at[pl.ds(ci * _CHUNK, _CHUNK)], src_v)
        pltpu.sync_copy(dst_hbm.at[pl.ds(ci * _CHUNK, _CHUNK)], dst_v)

        lane = lax.iota(jnp.int32, 16)

        def _do_group(gi):
            sid = src_v[pl.ds(gi * 16, 16)]
            did = dst_v[pl.ds(gi * 16, 16)]
            plsc.store_scatter(dup_v, [did], lane)
            wb = plsc.load_gather(dup_v, [did])
            collide = jnp.any(wb != lane)
            didx = [did + offs[r] for r in range(_RPT)]
            news = []
            for r in range(_RPT):
                v = plsc.load_gather(pool_v, [sid + offs[r]])
                cur = plsc.load_gather(acc_v, [didx[r]])
                new = jnp.maximum(cur, v)
                plsc.store_scatter(acc_v, [didx[r]], new)
                news.append(new)

            def _slow(_):
                def _retry_cond(f):
                    return f

                def _retry(_f):
                    f2 = jnp.bool_(False)
                    for r in range(_RPT):
                        chk = plsc.load_gather(acc_v, [didx[r]])
                        need = chk < news[r]
                        plsc.store_scatter(acc_v, [didx[r]], news[r],
                                           mask=need)
                        f2 = f2 | jnp.any(need)
                    return f2

                lax.while_loop(_retry_cond, _retry, jnp.bool_(True))
                return 0

            lax.cond(collide, _slow, lambda _: 0, 0)

        def _group(gi, c2):
            _do_group(2 * gi)
            _do_group(2 * gi + 1)
            return c2

        lax.fori_loop(0, _GROUPS // 2, _group, 0)
        return carry

    lax.fori_loop(0, _NCHUNK, _chunk, 0)

    pltpu.sync_copy(acc_v, out_hbm.at[pl.ds(base, _RPT * N)])


@functools.partial(
    pl.kernel,
    out_type=jax.ShapeDtypeStruct((H * N,), jnp.float32),
    mesh=plsc.VectorSubcoreMesh(core_axis_name="c", subcore_axis_name="s"),
    compiler_params=pltpu.CompilerParams(needs_layout_passes=False),
    scratch_types=[
        pltpu.VMEM((_RPT * N,), jnp.float32),
        pltpu.VMEM((_RPT * N,), jnp.float32),
        pltpu.VMEM((_CHUNK,), jnp.int32),
        pltpu.VMEM((_CHUNK,), jnp.int32),
        pltpu.VMEM((N,), jnp.int32),
    ],
)
def _sc_segmax_flat(p_hbm, src_hbm, dst_hbm, out_hbm, pool_v, acc_v, src_v,
                    dst_v, dup_v):
    _sc_segmax_body(p_hbm, src_hbm, dst_hbm, out_hbm, pool_v, acc_v,
                    src_v, dst_v, dup_v)


def _sc_segmax(p, src, dst):
    return _sc_segmax_flat(p.reshape(H * N), src, dst).reshape(H, N)


def _tc_first_body(s_ref, a_ref, wn_ref, b_ref, wp_ref, bp_ref,
                   g_ref, p_ref):
    g = jax.nn.relu(s_ref[...] + _dotT(wn_ref[...], a_ref[...])
                    + b_ref[...])
    g_ref[...] = g
    p_ref[...] = jax.nn.relu(_dotT(wp_ref[...], g) + bp_ref[...])


def _tc_mid_body(g_ref, a_ref, ws_ref, wn_ref, b_ref, wp_ref, bp_ref,
                 go_ref, p_ref):
    g = jax.nn.relu(_dotT(ws_ref[...], g_ref[...])
                    + _dotT(wn_ref[...], a_ref[...]) + b_ref[...])
    go_ref[...] = g
    p_ref[...] = jax.nn.relu(_dotT(wp_ref[...], g) + bp_ref[...])


def _tc_final_body(g_ref, a_ref, ws_ref, wn_ref, b_ref, out_ref):
    o = (_dotT(ws_ref[...], g_ref[...])
         + _dotT(wn_ref[...], a_ref[...]) + b_ref[...])
    out_ref[...] = o.T


def _tc_first(s, a, wn, b, wp, bp):
    return pl.pallas_call(
        _tc_first_body,
        out_shape=[jax.ShapeDtypeStruct((H, N), jnp.float32)] * 2,
    )(s, a, wn, b, wp, bp)


def _tc_mid(g, a, ws, wn, b, wp, bp):
    return pl.pallas_call(
        _tc_mid_body,
        out_shape=[jax.ShapeDtypeStruct((H, N), jnp.float32)] * 2,
    )(g, a, ws, wn, b, wp, bp)


def _tc_final(g, a, ws, wn, b):
    return pl.pallas_call(
        _tc_final_body,
        out_shape=jax.ShapeDtypeStruct((N, H), jnp.float32),
    )(g, a, ws, wn, b)


def kernel(edge_index, Wp1, bp1, Ws1, Wn1, b1, Wp2, bp2, Ws2, Wn2, b2):
    src = edge_index[0]
    dst = edge_index[1]
    pad = jnp.zeros((H, N - H), jnp.float32)

    p1 = jax.nn.relu(jnp.concatenate([Wp1.T, pad], axis=1) + bp1[:, None])
    s1 = jnp.concatenate([Ws1.T, pad], axis=1)

    a1 = _sc_segmax(p1, src, dst)
    g1, p2 = _tc_first(s1, a1, Wn1, b1[:, None], Wp1, bp1[:, None])
    a2 = _sc_segmax(p2, src, dst)
    g2, p3 = _tc_mid(g1, a2, Ws1, Wn1, b1[:, None], Wp2, bp2[:, None])
    a3 = _sc_segmax(p3, src, dst)
    return _tc_final(g2, a3, Ws2, Wn2, b2[:, None])


# speedup vs baseline: 2.1924x; 1.0212x over previous
import functools

import jax
import jax.numpy as jnp
from jax import lax
from jax.experimental import pallas as pl
from jax.experimental.pallas import tpu as pltpu
from jax.experimental.pallas import tpu_sc as plsc

N = 10000
E = 320000
H = 128

_NC = 2
_NS = 16
_NTILES = _NC * _NS
_RPT = H // _NTILES
_CHUNK = 8000
_NCHUNK = E // _CHUNK
_GROUPS = _CHUNK // 16
_BATCH = 5

_BLK = 1000
_NBLK = N // _BLK


def _dotT(w, x):
    return lax.dot_general(w, x, (((0,), (0,)), ((), ())),
                           preferred_element_type=jnp.float32)


def _sc_segmax_body(p_hbm, src_hbm, dst_hbm, out_hbm, pool_v, acc_v,
                    src_v, dst_v, dup_v):
    wid = lax.axis_index("s") * _NC + lax.axis_index("c")
    base = wid * _RPT * N

    pltpu.sync_copy(p_hbm.at[pl.ds(base, _RPT * N)], pool_v)

    zero16 = jnp.zeros((16,), jnp.float32)

    def _zero(i, carry):
        acc_v[pl.ds(i * 16, 16)] = zero16
        return carry

    lax.fori_loop(0, _RPT * N // 16, _zero, 0)

    offs = [jnp.full((16,), r * N, jnp.int32) for r in range(_RPT)]

    def _chunk(ci, carry):
        pltpu.sync_copy(src_hbm.at[pl.ds(ci * _CHUNK, _CHUNK)], src_v)
        pltpu.sync_copy(dst_hbm.at[pl.ds(ci * _CHUNK, _CHUNK)], dst_v)

        lane = lax.iota(jnp.int32, 16)

        def _blind_group(gi):
            sid = src_v[pl.ds(gi * 16, 16)]
            did = dst_v[pl.ds(gi * 16, 16)]
            plsc.store_scatter(dup_v, [did], lane)
            wb = plsc.load_gather(dup_v, [did])
            for r in range(_RPT):
                v = plsc.load_gather(pool_v, [sid + offs[r]])
                didx = did + offs[r]
                cur = plsc.load_gather(acc_v, [didx])
                plsc.store_scatter(acc_v, [didx], jnp.maximum(cur, v))
            return wb != lane

        def _careful_group(gi):
            sid = src_v[pl.ds(gi * 16, 16)]
            did = dst_v[pl.ds(gi * 16, 16)]
            vals = [plsc.load_gather(pool_v, [sid + offs[r]])
                    for r in range(_RPT)]
            didx = [did + offs[r] for r in range(_RPT)]

            def _retry_cond(f):
                return f

            def _retry(_f):
                f2 = jnp.bool_(False)
                for r in range(_RPT):
                    chk = plsc.load_gather(acc_v, [didx[r]])
                    need = chk < vals[r]
                    plsc.store_scatter(acc_v, [didx[r]], vals[r],
                                       mask=need)
                    f2 = f2 | jnp.any(need)
                return f2

            lax.while_loop(_retry_cond, _retry, jnp.bool_(True))

        def _batch(bi, c2):
            g0 = bi * _BATCH
            m = _blind_group(g0)
            for k in range(1, _BATCH):
                m = m | _blind_group(g0 + k)

            def _redo(_):
                for k in range(_BATCH):
                    _careful_group(g0 + k)
                return 0

            lax.cond(jnp.any(m), _redo, lambda _: 0, 0)
            return c2

        lax.fori_loop(0, _GROUPS // _BATCH, _batch, 0)
        return carry

    lax.fori_loop(0, _NCHUNK, _chunk, 0)

    pltpu.sync_copy(acc_v, out_hbm.at[pl.ds(base, _RPT * N)])


@functools.partial(
    pl.kernel,
    out_type=jax.ShapeDtypeStruct((H * N,), jnp.float32),
    mesh=plsc.VectorSubcoreMesh(core_axis_name="c", subcore_axis_name="s"),
    compiler_params=pltpu.CompilerParams(needs_layout_passes=False),
    scratch_types=[
        pltpu.VMEM((_RPT * N,), jnp.float32),
        pltpu.VMEM((_RPT * N,), jnp.float32),
        pltpu.VMEM((_CHUNK,), jnp.int32),
        pltpu.VMEM((_CHUNK,), jnp.int32),
        pltpu.VMEM((N,), jnp.int32),
    ],
)
def _sc_segmax_flat(p_hbm, src_hbm, dst_hbm, out_hbm, pool_v, acc_v, src_v,
                    dst_v, dup_v):
    _sc_segmax_body(p_hbm, src_hbm, dst_hbm, out_hbm, pool_v, acc_v,
                    src_v, dst_v, dup_v)


def _sc_segmax(p, src, dst):
    return _sc_segmax_flat(p.reshape(H * N), src, dst).reshape(H, N)


def _tc_first_body(s_ref, a_ref, wn_ref, b_ref, wp_ref, bp_ref,
                   g_ref, p_ref):
    g = jax.nn.relu(s_ref[...] + _dotT(wn_ref[...], a_ref[...])
                    + b_ref[...])
    g_ref[...] = g
    p_ref[...] = jax.nn.relu(_dotT(wp_ref[...], g) + bp_ref[...])


def _tc_mid_body(g_ref, a_ref, ws_ref, wn_ref, b_ref, wp_ref, bp_ref,
                 go_ref, p_ref):
    g = jax.nn.relu(_dotT(ws_ref[...], g_ref[...])
                    + _dotT(wn_ref[...], a_ref[...]) + b_ref[...])
    go_ref[...] = g
    p_ref[...] = jax.nn.relu(_dotT(wp_ref[...], g) + bp_ref[...])


def _tc_final_body(g_ref, a_ref, ws_ref, wn_ref, b_ref, out_ref):
    o = (_dotT(ws_ref[...], g_ref[...])
         + _dotT(wn_ref[...], a_ref[...]) + b_ref[...])
    out_ref[...] = o.T


def _tc_first(s, a, wn, b, wp, bp):
    return pl.pallas_call(
        _tc_first_body,
        out_shape=[jax.ShapeDtypeStruct((H, N), jnp.float32)] * 2,
    )(s, a, wn, b, wp, bp)


def _tc_mid(g, a, ws, wn, b, wp, bp):
    return pl.pallas_call(
        _tc_mid_body,
        out_shape=[jax.ShapeDtypeStruct((H, N), jnp.float32)] * 2,
    )(g, a, ws, wn, b, wp, bp)


def _tc_final(g, a, ws, wn, b):
    return pl.pallas_call(
        _tc_final_body,
        out_shape=jax.ShapeDtypeStruct((N, H), jnp.float32),
    )(g, a, ws, wn, b)


def kernel(edge_index, Wp1, bp1, Ws1, Wn1, b1, Wp2, bp2, Ws2, Wn2, b2):
    src = edge_index[0]
    dst = edge_index[1]
    pad = jnp.zeros((H, N - H), jnp.float32)

    p1 = jax.nn.relu(jnp.concatenate([Wp1.T, pad], axis=1) + bp1[:, None])
    s1 = jnp.concatenate([Ws1.T, pad], axis=1)

    a1 = _sc_segmax(p1, src, dst)
    g1, p2 = _tc_first(s1, a1, Wn1, b1[:, None], Wp1, bp1[:, None])
    a2 = _sc_segmax(p2, src, dst)
    g2, p3 = _tc_mid(g1, a2, Ws1, Wn1, b1[:, None], Wp2, bp2[:, None])
    a3 = _sc_segmax(p3, src, dst)
    return _tc_final(g2, a3, Ws2, Wn2, b2[:, None])


# speedup vs baseline: 2.1954x; 1.0014x over previous
import functools

import jax
import jax.numpy as jnp
from jax import lax
from jax.experimental import pallas as pl
from jax.experimental.pallas import tpu as pltpu
from jax.experimental.pallas import tpu_sc as plsc

N = 10000
E = 320000
H = 128

_NC = 2
_NS = 16
_NTILES = _NC * _NS
_RPT = H // _NTILES
_CHUNK = 8000
_NCHUNK = E // _CHUNK
_GROUPS = _CHUNK // 16
_BATCH = 5

_BLK = 1000
_NBLK = N // _BLK


def _dotT(w, x):
    return lax.dot_general(w, x, (((0,), (0,)), ((), ())),
                           preferred_element_type=jnp.float32)


def _sc_segmax_body(p_hbm, src_hbm, dst_hbm, out_hbm, pool_vs, acc_vs,
                    src_v, dst_v, dup_v):
    wid = lax.axis_index("s") * _NC + lax.axis_index("c")
    base = wid * _RPT * N

    for r in range(_RPT):
        pltpu.sync_copy(p_hbm.at[pl.ds(base + r * N, N)], pool_vs[r])

    zero16 = jnp.zeros((16,), jnp.float32)

    def _zero(i, carry):
        for r in range(_RPT):
            acc_vs[r][pl.ds(i * 16, 16)] = zero16
        return carry

    lax.fori_loop(0, N // 16, _zero, 0)

    def _chunk(ci, carry):
        pltpu.sync_copy(src_hbm.at[pl.ds(ci * _CHUNK, _CHUNK)], src_v)
        pltpu.sync_copy(dst_hbm.at[pl.ds(ci * _CHUNK, _CHUNK)], dst_v)

        lane = lax.iota(jnp.int32, 16)

        def _blind_group(gi):
            sid = src_v[pl.ds(gi * 16, 16)]
            did = dst_v[pl.ds(gi * 16, 16)]
            plsc.store_scatter(dup_v, [did], lane)
            wb = plsc.load_gather(dup_v, [did])
            for r in range(_RPT):
                v = plsc.load_gather(pool_vs[r], [sid])
                cur = plsc.load_gather(acc_vs[r], [did])
                plsc.store_scatter(acc_vs[r], [did], jnp.maximum(cur, v))
            return wb != lane

        def _careful_group(gi):
            sid = src_v[pl.ds(gi * 16, 16)]
            did = dst_v[pl.ds(gi * 16, 16)]
            vals = [plsc.load_gather(pool_vs[r], [sid])
                    for r in range(_RPT)]

            def _retry_cond(f):
                return f

            def _retry(_f):
                f2 = jnp.bool_(False)
                for r in range(_RPT):
                    chk = plsc.load_gather(acc_vs[r], [did])
                    need = chk < vals[r]
                    plsc.store_scatter(acc_vs[r], [did], vals[r],
                                       mask=need)
                    f2 = f2 | jnp.any(need)
                return f2

            lax.while_loop(_retry_cond, _retry, jnp.bool_(True))

        def _batch(bi, c2):
            g0 = bi * _BATCH
            m = _blind_group(g0)
            for k in range(1, _BATCH):
                m = m | _blind_group(g0 + k)

            def _redo(_):
                for k in range(_BATCH):
                    _careful_group(g0 + k)
                return 0

            lax.cond(jnp.any(m), _redo, lambda _: 0, 0)
            return c2

        lax.fori_loop(0, _GROUPS // _BATCH, _batch, 0)
        return carry

    lax.fori_loop(0, _NCHUNK, _chunk, 0)

    for r in range(_RPT):
        pltpu.sync_copy(acc_vs[r], out_hbm.at[pl.ds(base + r * N, N)])


@functools.partial(
    pl.kernel,
    out_type=jax.ShapeDtypeStruct((H * N,), jnp.float32),
    mesh=plsc.VectorSubcoreMesh(core_axis_name="c", subcore_axis_name="s"),
    compiler_params=pltpu.CompilerParams(needs_layout_passes=False),
    scratch_types=[
        pltpu.VMEM((N,), jnp.float32),
        pltpu.VMEM((N,), jnp.float32),
        pltpu.VMEM((N,), jnp.float32),
        pltpu.VMEM((N,), jnp.float32),
        pltpu.VMEM((N,), jnp.float32),
        pltpu.VMEM((N,), jnp.float32),
        pltpu.VMEM((N,), jnp.float32),
        pltpu.VMEM((N,), jnp.float32),
        pltpu.VMEM((_CHUNK,), jnp.int32),
        pltpu.VMEM((_CHUNK,), jnp.int32),
        pltpu.VMEM((N,), jnp.int32),
    ],
)
def _sc_segmax_flat(p_hbm, src_hbm, dst_hbm, out_hbm, p0, p1, p2, p3,
                    a0, a1, a2, a3, src_v, dst_v, dup_v):
    _sc_segmax_body(p_hbm, src_hbm, dst_hbm, out_hbm, (p0, p1, p2, p3),
                    (a0, a1, a2, a3), src_v, dst_v, dup_v)


def _sc_segmax(p, src, dst):
    return _sc_segmax_flat(p.reshape(H * N), src, dst).reshape(H, N)


def _tc_first_body(s_ref, a_ref, wn_ref, b_ref, wp_ref, bp_ref,
                   g_ref, p_ref):
    g = jax.nn.relu(s_ref[...] + _dotT(wn_ref[...], a_ref[...])
                    + b_ref[...])
    g_ref[...] = g
    p_ref[...] = jax.nn.relu(_dotT(wp_ref[...], g) + bp_ref[...])


def _tc_mid_body(g_ref, a_ref, ws_ref, wn_ref, b_ref, wp_ref, bp_ref,
                 go_ref, p_ref):
    g = jax.nn.relu(_dotT(ws_ref[...], g_ref[...])
                    + _dotT(wn_ref[...], a_ref[...]) + b_ref[...])
    go_ref[...] = g
    p_ref[...] = jax.nn.relu(_dotT(wp_ref[...], g) + bp_ref[...])


def _tc_final_body(g_ref, a_ref, ws_ref, wn_ref, b_ref, out_ref):
    o = (_dotT(ws_ref[...], g_ref[...])
         + _dotT(wn_ref[...], a_ref[...]) + b_ref[...])
    out_ref[...] = o.T


def _tc_first(s, a, wn, b, wp, bp):
    return pl.pallas_call(
        _tc_first_body,
        out_shape=[jax.ShapeDtypeStruct((H, N), jnp.float32)] * 2,
    )(s, a, wn, b, wp, bp)


def _tc_mid(g, a, ws, wn, b, wp, bp):
    return pl.pallas_call(
        _tc_mid_body,
        out_shape=[jax.ShapeDtypeStruct((H, N), jnp.float32)] * 2,
    )(g, a, ws, wn, b, wp, bp)


def _tc_final(g, a, ws, wn, b):
    return pl.pallas_call(
        _tc_final_body,
        out_shape=jax.ShapeDtypeStruct((N, H), jnp.float32),
    )(g, a, ws, wn, b)


def kernel(edge_index, Wp1, bp1, Ws1, Wn1, b1, Wp2, bp2, Ws2, Wn2, b2):
    src = edge_index[0]
    dst = edge_index[1]
    pad = jnp.zeros((H, N - H), jnp.float32)

    p1 = jax.nn.relu(jnp.concatenate([Wp1.T, pad], axis=1) + bp1[:, None])
    s1 = jnp.concatenate([Ws1.T, pad], axis=1)

    a1 = _sc_segmax(p1, src, dst)
    g1, p2 = _tc_first(s1, a1, Wn1, b1[:, None], Wp1, bp1[:, None])
    a2 = _sc_segmax(p2, src, dst)
    g2, p3 = _tc_mid(g1, a2, Ws1, Wn1, b1[:, None], Wp2, bp2[:, None])
    a3 = _sc_segmax(p3, src, dst)
    return _tc_final(g2, a3, Ws2, Wn2, b2[:, None])


# speedup vs baseline: 2.7116x; 1.2351x over previous
import functools

import jax
import jax.numpy as jnp
from jax import lax
from jax.experimental import pallas as pl
from jax.experimental.pallas import tpu as pltpu
from jax.experimental.pallas import tpu_sc as plsc

N = 10000
E = 320000
H = 128

_NC = 2
_NS = 16
_NTILES = _NC * _NS
_RPT = H // _NTILES
_CHUNK = 8000
_NCHUNK = E // _CHUNK
_GROUPS = _CHUNK // 16
_BATCH = 5

_BLK = 1000
_NBLK = N // _BLK


def _dotT(w, x):
    return lax.dot_general(w, x, (((0,), (0,)), ((), ())),
                           preferred_element_type=jnp.float32)


def _sc_segmax_body(p_hbm, src_hbm, dst_hbm, out_hbm, pool_vs, acc_vs,
                    src_v, dst_v, dup_v):
    wid = lax.axis_index("s") * _NC + lax.axis_index("c")
    base = wid * _RPT * N

    for r in range(_RPT):
        pltpu.sync_copy(p_hbm.at[pl.ds(base + r * N, N)], pool_vs[r])

    zero16 = jnp.zeros((16,), jnp.float32)

    def _zero(i, carry):
        for r in range(_RPT):
            acc_vs[r][pl.ds(i * 16, 16)] = zero16
        return carry

    lax.fori_loop(0, N // 16, _zero, 0)

    def _chunk(ci, carry):
        pltpu.sync_copy(src_hbm.at[pl.ds(ci * _CHUNK, _CHUNK)], src_v)
        pltpu.sync_copy(dst_hbm.at[pl.ds(ci * _CHUNK, _CHUNK)], dst_v)

        lane = lax.iota(jnp.int32, 16)

        def _blind_group(gi):
            sid = src_v[pl.ds(gi * 16, 16)]
            did = dst_v[pl.ds(gi * 16, 16)]
            plsc.store_scatter(dup_v, [did], lane)
            wb = plsc.load_gather(dup_v, [did])
            vs = [plsc.load_gather(pool_vs[r], [sid]) for r in range(_RPT)]
            curs = [plsc.load_gather(acc_vs[r], [did]) for r in range(_RPT)]
            news = [jnp.maximum(curs[r], vs[r]) for r in range(_RPT)]
            for r in range(_RPT):
                plsc.store_scatter(acc_vs[r], [did], news[r])
            return wb != lane

        def _careful_group(gi):
            sid = src_v[pl.ds(gi * 16, 16)]
            did = dst_v[pl.ds(gi * 16, 16)]
            vals = [plsc.load_gather(pool_vs[r], [sid])
                    for r in range(_RPT)]

            def _retry_cond(f):
                return f

            def _retry(_f):
                f2 = jnp.bool_(False)
                for r in range(_RPT):
                    chk = plsc.load_gather(acc_vs[r], [did])
                    need = chk < vals[r]
                    plsc.store_scatter(acc_vs[r], [did], vals[r],
                                       mask=need)
                    f2 = f2 | jnp.any(need)
                return f2

            lax.while_loop(_retry_cond, _retry, jnp.bool_(True))

        def _batch(bi, c2):
            g0 = bi * _BATCH
            m = _blind_group(g0)
            for k in range(1, _BATCH):
                m = m | _blind_group(g0 + k)

            def _redo(_):
                for k in range(_BATCH):
                    _careful_group(g0 + k)
                return 0

            lax.cond(jnp.any(m), _redo, lambda _: 0, 0)
            return c2

        lax.fori_loop(0, _GROUPS // _BATCH, _batch, 0)
        return carry

    lax.fori_loop(0, _NCHUNK, _chunk, 0)

    for r in range(_RPT):
        pltpu.sync_copy(acc_vs[r], out_hbm.at[pl.ds(base + r * N, N)])


@functools.partial(
    pl.kernel,
    out_type=jax.ShapeDtypeStruct((H * N,), jnp.float32),
    mesh=plsc.VectorSubcoreMesh(core_axis_name="c", subcore_axis_name="s"),
    compiler_params=pltpu.CompilerParams(needs_layout_passes=False),
    scratch_types=[
        pltpu.VMEM((N,), jnp.float32),
        pltpu.VMEM((N,), jnp.float32),
        pltpu.VMEM((N,), jnp.float32),
        pltpu.VMEM((N,), jnp.float32),
        pltpu.VMEM((N,), jnp.float32),
        pltpu.VMEM((N,), jnp.float32),
        pltpu.VMEM((N,), jnp.float32),
        pltpu.VMEM((N,), jnp.float32),
        pltpu.VMEM((_CHUNK,), jnp.int32),
        pltpu.VMEM((_CHUNK,), jnp.int32),
        pltpu.VMEM((N,), jnp.int32),
    ],
)
def _sc_segmax_flat(p_hbm, src_hbm, dst_hbm, out_hbm, p0, p1, p2, p3,
                    a0, a1, a2, a3, src_v, dst_v, dup_v):
    _sc_segmax_body(p_hbm, src_hbm, dst_hbm, out_hbm, (p0, p1, p2, p3),
                    (a0, a1, a2, a3), src_v, dst_v, dup_v)


def _sc_segmax(p, src, dst):
    return _sc_segmax_flat(p.reshape(H * N), src, dst).reshape(H, N)


def _tc_first_body(s_ref, a_ref, wn_ref, b_ref, wp_ref, bp_ref,
                   g_ref, p_ref):
    g = jax.nn.relu(s_ref[...] + _dotT(wn_ref[...], a_ref[...])
                    + b_ref[...])
    g_ref[...] = g
    p_ref[...] = jax.nn.relu(_dotT(wp_ref[...], g) + bp_ref[...])


def _tc_mid_body(g_ref, a_ref, ws_ref, wn_ref, b_ref, wp_ref, bp_ref,
                 go_ref, p_ref):
    g = jax.nn.relu(_dotT(ws_ref[...], g_ref[...])
                    + _dotT(wn_ref[...], a_ref[...]) + b_ref[...])
    go_ref[...] = g
    p_ref[...] = jax.nn.relu(_dotT(wp_ref[...], g) + bp_ref[...])


def _tc_final_body(g_ref, a_ref, ws_ref, wn_ref, b_ref, out_ref):
    o = (_dotT(ws_ref[...], g_ref[...])
         + _dotT(wn_ref[...], a_ref[...]) + b_ref[...])
    out_ref[...] = o.T


def _tc_first(s, a, wn, b, wp, bp):
    return pl.pallas_call(
        _tc_first_body,
        out_shape=[jax.ShapeDtypeStruct((H, N), jnp.float32)] * 2,
    )(s, a, wn, b, wp, bp)


def _tc_mid(g, a, ws, wn, b, wp, bp):
    return pl.pallas_call(
        _tc_mid_body,
        out_shape=[jax.ShapeDtypeStruct((H, N), jnp.float32)] * 2,
    )(g, a, ws, wn, b, wp, bp)


def _tc_final(g, a, ws, wn, b):
    return pl.pallas_call(
        _tc_final_body,
        out_shape=jax.ShapeDtypeStruct((N, H), jnp.float32),
    )(g, a, ws, wn, b)


def kernel(edge_index, Wp1, bp1, Ws1, Wn1, b1, Wp2, bp2, Ws2, Wn2, b2):
    src = edge_index[0]
    dst = edge_index[1]
    pad = jnp.zeros((H, N - H), jnp.float32)

    p1 = jax.nn.relu(jnp.concatenate([Wp1.T, pad], axis=1) + bp1[:, None])
    s1 = jnp.concatenate([Ws1.T, pad], axis=1)

    a1 = _sc_segmax(p1, src, dst)
    g1, p2 = _tc_first(s1, a1, Wn1, b1[:, None], Wp1, bp1[:, None])
    a2 = _sc_segmax(p2, src, dst)
    g2, p3 = _tc_mid(g1, a2, Ws1, Wn1, b1[:, None], Wp2, bp2[:, None])
    a3 = _sc_segmax(p3, src, dst)
    return _tc_final(g2, a3, Ws2, Wn2, b2[:, None])


# speedup vs baseline: 3.0667x; 1.1310x over previous
import functools

import jax
import jax.numpy as jnp
from jax import lax
from jax.experimental import pallas as pl
from jax.experimental.pallas import tpu as pltpu
from jax.experimental.pallas import tpu_sc as plsc

N = 10000
E = 320000
H = 128

_NC = 2
_NS = 16
_NTILES = _NC * _NS
_RPT = H // _NTILES
_CHUNK = 8000
_NCHUNK = E // _CHUNK
_GROUPS = _CHUNK // 16
_BATCH = 5

_BLK = 1000
_NBLK = N // _BLK


def _dotT(w, x):
    return lax.dot_general(w, x, (((0,), (0,)), ((), ())),
                           preferred_element_type=jnp.float32)


def _sc_segmax_body(p_hbm, src_hbm, dst_hbm, out_hbm, pool_vs, acc_vs,
                    src_v, dst_v, dup_v):
    wid = lax.axis_index("s") * _NC + lax.axis_index("c")
    base = wid * _RPT * N

    for r in range(_RPT):
        pltpu.sync_copy(p_hbm.at[pl.ds(base + r * N, N)], pool_vs[r])

    zero16 = jnp.zeros((16,), jnp.float32)

    def _zero(i, carry):
        for r in range(_RPT):
            acc_vs[r][pl.ds(i * 16, 16)] = zero16
        return carry

    lax.fori_loop(0, N // 16, _zero, 0)

    def _chunk(ci, carry):
        pltpu.sync_copy(src_hbm.at[pl.ds(ci * _CHUNK, _CHUNK)], src_v)
        pltpu.sync_copy(dst_hbm.at[pl.ds(ci * _CHUNK, _CHUNK)], dst_v)

        lane = lax.iota(jnp.int32, 16)

        def _blind_group(sid, did):
            plsc.store_scatter(dup_v, [did], lane)
            wb = plsc.load_gather(dup_v, [did])
            vs = [plsc.load_gather(pool_vs[r], [sid]) for r in range(_RPT)]
            curs = [plsc.load_gather(acc_vs[r], [did]) for r in range(_RPT)]
            news = [jnp.maximum(curs[r], vs[r]) for r in range(_RPT)]
            for r in range(_RPT):
                plsc.store_scatter(acc_vs[r], [did], news[r])
            return wb != lane

        def _careful_group(gi):
            sid = src_v[pl.ds(gi * 16, 16)]
            did = dst_v[pl.ds(gi * 16, 16)]
            vals = [plsc.load_gather(pool_vs[r], [sid])
                    for r in range(_RPT)]

            def _retry_cond(f):
                return f

            def _retry(_f):
                f2 = jnp.bool_(False)
                for r in range(_RPT):
                    chk = plsc.load_gather(acc_vs[r], [did])
                    need = chk < vals[r]
                    plsc.store_scatter(acc_vs[r], [did], vals[r],
                                       mask=need)
                    f2 = f2 | jnp.any(need)
                return f2

            lax.while_loop(_retry_cond, _retry, jnp.bool_(True))

        def _batch(bi, c2):
            g0 = bi * _BATCH
            sids = [src_v[pl.ds((g0 + k) * 16, 16)] for k in range(_BATCH)]
            dids = [dst_v[pl.ds((g0 + k) * 16, 16)] for k in range(_BATCH)]
            m = _blind_group(sids[0], dids[0])
            for k in range(1, _BATCH):
                m = m | _blind_group(sids[k], dids[k])

            def _redo(_):
                for k in range(_BATCH):
                    _careful_group(g0 + k)
                return 0

            lax.cond(jnp.any(m), _redo, lambda _: 0, 0)
            return c2

        lax.fori_loop(0, _GROUPS // _BATCH, _batch, 0)
        return carry

    lax.fori_loop(0, _NCHUNK, _chunk, 0)

    for r in range(_RPT):
        pltpu.sync_copy(acc_vs[r], out_hbm.at[pl.ds(base + r * N, N)])


@functools.partial(
    pl.kernel,
    out_type=jax.ShapeDtypeStruct((H * N,), jnp.float32),
    mesh=plsc.VectorSubcoreMesh(core_axis_name="c", subcore_axis_name="s"),
    compiler_params=pltpu.CompilerParams(needs_layout_passes=False),
    scratch_types=[
        pltpu.VMEM((N,), jnp.float32),
        pltpu.VMEM((N,), jnp.float32),
        pltpu.VMEM((N,), jnp.float32),
        pltpu.VMEM((N,), jnp.float32),
        pltpu.VMEM((N,), jnp.float32),
        pltpu.VMEM((N,), jnp.float32),
        pltpu.VMEM((N,), jnp.float32),
        pltpu.VMEM((N,), jnp.float32),
        pltpu.VMEM((_CHUNK,), jnp.int32),
        pltpu.VMEM((_CHUNK,), jnp.int32),
        pltpu.VMEM((N,), jnp.int32),
    ],
)
def _sc_segmax_flat(p_hbm, src_hbm, dst_hbm, out_hbm, p0, p1, p2, p3,
                    a0, a1, a2, a3, src_v, dst_v, dup_v):
    _sc_segmax_body(p_hbm, src_hbm, dst_hbm, out_hbm, (p0, p1, p2, p3),
                    (a0, a1, a2, a3), src_v, dst_v, dup_v)


def _sc_segmax(p, src, dst):
    return _sc_segmax_flat(p.reshape(H * N), src, dst).reshape(H, N)


def _tc_first_body(s_ref, a_ref, wn_ref, b_ref, wp_ref, bp_ref,
                   g_ref, p_ref):
    g = jax.nn.relu(s_ref[...] + _dotT(wn_ref[...], a_ref[...])
                    + b_ref[...])
    g_ref[...] = g
    p_ref[...] = jax.nn.relu(_dotT(wp_ref[...], g) + bp_ref[...])


def _tc_mid_body(g_ref, a_ref, ws_ref, wn_ref, b_ref, wp_ref, bp_ref,
                 go_ref, p_ref):
    g = jax.nn.relu(_dotT(ws_ref[...], g_ref[...])
                    + _dotT(wn_ref[...], a_ref[...]) + b_ref[...])
    go_ref[...] = g
    p_ref[...] = jax.nn.relu(_dotT(wp_ref[...], g) + bp_ref[...])


def _tc_final_body(g_ref, a_ref, ws_ref, wn_ref, b_ref, out_ref):
    o = (_dotT(ws_ref[...], g_ref[...])
         + _dotT(wn_ref[...], a_ref[...]) + b_ref[...])
    out_ref[...] = o.T


def _tc_first(s, a, wn, b, wp, bp):
    return pl.pallas_call(
        _tc_first_body,
        out_shape=[jax.ShapeDtypeStruct((H, N), jnp.float32)] * 2,
    )(s, a, wn, b, wp, bp)


def _tc_mid(g, a, ws, wn, b, wp, bp):
    return pl.pallas_call(
        _tc_mid_body,
        out_shape=[jax.ShapeDtypeStruct((H, N), jnp.float32)] * 2,
    )(g, a, ws, wn, b, wp, bp)


def _tc_final(g, a, ws, wn, b):
    return pl.pallas_call(
        _tc_final_body,
        out_shape=jax.ShapeDtypeStruct((N, H), jnp.float32),
    )(g, a, ws, wn, b)


def kernel(edge_index, Wp1, bp1, Ws1, Wn1, b1, Wp2, bp2, Ws2, Wn2, b2):
    src = edge_index[0]
    dst = edge_index[1]
    pad = jnp.zeros((H, N - H), jnp.float32)

    p1 = jax.nn.relu(jnp.concatenate([Wp1.T, pad], axis=1) + bp1[:, None])
    s1 = jnp.concatenate([Ws1.T, pad], axis=1)

    a1 = _sc_segmax(p1, src, dst)
    g1, p2 = _tc_first(s1, a1, Wn1, b1[:, None], Wp1, bp1[:, None])
    a2 = _sc_segmax(p2, src, dst)
    g2, p3 = _tc_mid(g1, a2, Ws1, Wn1, b1[:, None], Wp2, bp2[:, None])
    a3 = _sc_segmax(p3, src, dst)
    return _tc_final(g2, a3, Ws2, Wn2, b2[:, None])


# speedup vs baseline: 7.4199x; 2.4195x over previous
import functools

import jax
import jax.numpy as jnp
from jax import lax
from jax.experimental import pallas as pl
from jax.experimental.pallas import tpu as pltpu
from jax.experimental.pallas import tpu_sc as plsc

N = 10000
E = 320000
H = 128
_HP = H // 2

_NC = 2
_NS = 16
_NPR = _HP // _NS
_EHALF = E // _NC
_CHUNK = 8000
_NCHUNK = _EHALF // _CHUNK
_GROUPS = _CHUNK // 16
_BATCH = 5


def _dotT(w, x):
    return lax.dot_general(w, x, (((0,), (0,)), ((), ())),
                           preferred_element_type=jnp.float32)


def _bmax(a, b):
    am = plsc.bitcast(a, jnp.bfloat16)
    bm = plsc.bitcast(b, jnp.bfloat16)
    return plsc.bitcast(jnp.maximum(am, bm), jnp.int32)


def _sc_segmax_body(p_hbm, edge_hbm, out_hbm, pool_vs, acc_vs, edge0,
                    edge1, sem0, sem1):
    sub = lax.axis_index("s")
    core = lax.axis_index("c")
    base = sub * _NPR * N
    obase = core * (_HP * N) + base
    ebase = core * _EHALF

    for r in range(_NPR):
        pltpu.sync_copy(p_hbm.at[pl.ds(base + r * N, N)], pool_vs[r])

    zero16 = jnp.zeros((16,), jnp.int32)

    def _zero(i, carry):
        for r in range(_NPR):
            acc_vs[r][pl.ds(i * 16, 16)] = zero16
        return carry

    lax.fori_loop(0, N // 16, _zero, 0)

    def _blind_group(sid, did):
        _, lastm = plsc.scan_count(did)
        vs = [plsc.load_gather(pool_vs[r], [sid]) for r in range(_NPR)]
        curs = [plsc.load_gather(acc_vs[r], [did]) for r in range(_NPR)]
        news = [_bmax(curs[r], vs[r]) for r in range(_NPR)]
        for r in range(_NPR):
            plsc.store_scatter(acc_vs[r], [did], news[r])
        return ~lastm

    def _careful_group(sid, did):
        vals = [plsc.load_gather(pool_vs[r], [sid])
                for r in range(_NPR)]

        def _retry_cond(f):
            return f

        def _retry(_f):
            f2 = jnp.bool_(False)
            for r in range(_NPR):
                chk = plsc.load_gather(acc_vs[r], [did])
                want = _bmax(chk, vals[r])
                need = want != chk
                plsc.store_scatter(acc_vs[r], [did], want,
                                   mask=need)
                f2 = f2 | jnp.any(need)
            return f2

        lax.while_loop(_retry_cond, _retry, jnp.bool_(True))

    def _process(edge_v):
        def _batch(bi, c2):
            g0 = bi * _BATCH
            es = [edge_v[pl.ds((g0 + k) * 16, 16)] for k in range(_BATCH)]
            sids = [e & 0x3FFF for e in es]
            dids = [lax.shift_right_logical(e, 14) for e in es]
            m = _blind_group(sids[0], dids[0])
            for k in range(1, _BATCH):
                m = m | _blind_group(sids[k], dids[k])

            def _redo(_):
                for k in range(_BATCH):
                    _careful_group(sids[k], dids[k])
                return 0

            lax.cond(jnp.any(m), _redo, lambda _: 0, 0)
            return c2

        lax.fori_loop(0, _GROUPS // _BATCH, _batch, 0)

    def _start(ci, buf, sem):
        pltpu.make_async_copy(
            edge_hbm.at[pl.ds(ebase + ci * _CHUNK, _CHUNK)], buf,
            sem).start()

    def _wait(buf, sem):
        pltpu.make_async_copy(edge_hbm.at[pl.ds(0, _CHUNK)], buf,
                              sem).wait()

    _start(0, edge0, sem0)

    def _chunk_pair(cp, carry):
        ci0 = 2 * cp
        _start(ci0 + 1, edge1, sem1)
        _wait(edge0, sem0)
        _process(edge0)

        @pl.when(ci0 + 2 < _NCHUNK)
        def _():
            _start(ci0 + 2, edge0, sem0)

        _wait(edge1, sem1)
        _process(edge1)
        return carry

    lax.fori_loop(0, _NCHUNK // 2, _chunk_pair, 0)

    for r in range(_NPR):
        pltpu.sync_copy(acc_vs[r], out_hbm.at[pl.ds(obase + r * N, N)])


@functools.partial(
    pl.kernel,
    out_type=jax.ShapeDtypeStruct((_NC * _HP * N,), jnp.int32),
    mesh=plsc.VectorSubcoreMesh(core_axis_name="c", subcore_axis_name="s"),
    compiler_params=pltpu.CompilerParams(needs_layout_passes=False),
    scratch_types=[
        pltpu.VMEM((N,), jnp.int32),
        pltpu.VMEM((N,), jnp.int32),
        pltpu.VMEM((N,), jnp.int32),
        pltpu.VMEM((N,), jnp.int32),
        pltpu.VMEM((N,), jnp.int32),
        pltpu.VMEM((N,), jnp.int32),
        pltpu.VMEM((N,), jnp.int32),
        pltpu.VMEM((N,), jnp.int32),
        pltpu.VMEM((_CHUNK,), jnp.int32),
        pltpu.VMEM((_CHUNK,), jnp.int32),
        pltpu.SemaphoreType.DMA,
        pltpu.SemaphoreType.DMA,
    ],
)
def _sc_segmax_flat(p_hbm, edge_hbm, out_hbm, p0, p1, p2, p3,
                    a0, a1, a2, a3, edge0, edge1, sem0, sem1):
    _sc_segmax_body(p_hbm, edge_hbm, out_hbm, (p0, p1, p2, p3),
                    (a0, a1, a2, a3), edge0, edge1, sem0, sem1)


def _sc_segmax(packed, edges):
    return _sc_segmax_flat(packed, edges).reshape(_NC, _HP, N)


def _unpack_merge(a_ref):
    a0 = a_ref[0]
    a1 = a_ref[1]
    lo = jnp.maximum(a0 & 0xFFFF, a1 & 0xFFFF)
    hi = jnp.maximum(lax.shift_right_logical(a0, 16),
                     lax.shift_right_logical(a1, 16))
    flo = lax.bitcast_convert_type(lo << 16, jnp.float32)
    fhi = lax.bitcast_convert_type(hi << 16, jnp.float32)
    return jnp.concatenate([flo, fhi], axis=0)


def _pack(p):
    b = p.astype(jnp.bfloat16)
    bits = lax.bitcast_convert_type(b, jnp.uint16).astype(jnp.int32)
    return bits[:_HP] | (bits[_HP:] << 16)


def _tc_first_body(s_ref, a_ref, wn_ref, b_ref, wp_ref, bp_ref,
                   g_ref, p_ref):
    a = _unpack_merge(a_ref)
    g = jax.nn.relu(s_ref[...] + _dotT(wn_ref[...], a) + b_ref[...])
    g_ref[...] = g
    p_ref[...] = _pack(jax.nn.relu(_dotT(wp_ref[...], g) + bp_ref[...]))


def _tc_mid_body(g_ref, a_ref, ws_ref, wn_ref, b_ref, wp_ref, bp_ref,
                 go_ref, p_ref):
    a = _unpack_merge(a_ref)
    g = jax.nn.relu(_dotT(ws_ref[...], g_ref[...])
                    + _dotT(wn_ref[...], a) + b_ref[...])
    go_ref[...] = g
    p_ref[...] = _pack(jax.nn.relu(_dotT(wp_ref[...], g) + bp_ref[...]))


def _tc_final_body(g_ref, a_ref, ws_ref, wn_ref, b_ref, out_ref):
    a = _unpack_merge(a_ref)
    o = (_dotT(ws_ref[...], g_ref[...])
         + _dotT(wn_ref[...], a) + b_ref[...])
    out_ref[...] = o.T


def _tc_first(s, a, wn, b, wp, bp):
    return pl.pallas_call(
        _tc_first_body,
        out_shape=[jax.ShapeDtypeStruct((H, N), jnp.float32),
                   jax.ShapeDtypeStruct((_HP, N), jnp.int32)],
    )(s, a, wn, b, wp, bp)


def _tc_mid(g, a, ws, wn, b, wp, bp):
    return pl.pallas_call(
        _tc_mid_body,
        out_shape=[jax.ShapeDtypeStruct((H, N), jnp.float32),
                   jax.ShapeDtypeStruct((_HP, N), jnp.int32)],
    )(g, a, ws, wn, b, wp, bp)


def _tc_final(g, a, ws, wn, b):
    return pl.pallas_call(
        _tc_final_body,
        out_shape=jax.ShapeDtypeStruct((N, H), jnp.float32),
    )(g, a, ws, wn, b)


def kernel(edge_index, Wp1, bp1, Ws1, Wn1, b1, Wp2, bp2, Ws2, Wn2, b2):
    edges = edge_index[1] * 16384 + edge_index[0]
    pad = jnp.zeros((H, N - H), jnp.float32)

    p1 = jax.nn.relu(jnp.concatenate([Wp1.T, pad], axis=1) + bp1[:, None])
    s1 = jnp.concatenate([Ws1.T, pad], axis=1)
    p1p = _pack(p1).reshape(_HP * N)

    a1 = _sc_segmax(p1p, edges)
    g1, p2 = _tc_first(s1, a1, Wn1, b1[:, None], Wp1, bp1[:, None])
    a2 = _sc_segmax(p2.reshape(_HP * N), edges)
    g2, p3 = _tc_mid(g1, a2, Ws1, Wn1, b1[:, None], Wp2, bp2[:, None])
    a3 = _sc_segmax(p3.reshape(_HP * N), edges)
    return _tc_final(g2, a3, Ws2, Wn2, b2[:, None])
